# Initial kernel scaffold; baseline (speedup 1.0000x reference)
import jax
import jax.numpy as jnp
from jax.experimental import pallas as pl


def kernel(input, target, cluster_weight, cluster_bias, proj0, proj1, proj2, w0, b0, w1, b1, w2, b2):
    raise NotImplementedError("write your pallas kernel here")


# speedup vs baseline: 1.4499x; 1.4499x over previous
import functools

import jax
import jax.numpy as jnp
from jax.experimental import pallas as pl
from jax.experimental.pallas import tpu as pltpu

_CUT0 = 20000
_CUT1 = 60000
_VOCAB = 100000


def _flash_nll_body(x_ref, proj_ref, w_ref, b_ref, tgt_ref, out_ref,
                    ph, m, s, t, *, n_valid, cb, ncb, lo, hi):
    j = pl.program_id(1)

    @pl.when(j == 0)
    def _init():
        ph[...] = jnp.dot(x_ref[...], proj_ref[...],
                          preferred_element_type=jnp.float32)
        m[...] = jnp.full_like(m, -1e30)
        s[...] = jnp.zeros_like(s)
        t[...] = jnp.zeros_like(t)

    tcol = tgt_ref[:, :1]
    if lo is None:
        idx = jnp.where(tcol >= _CUT1, _CUT0,
                        jnp.where(tcol >= _CUT0, _CUT0 + 1, tcol))
    else:
        idx = jnp.where((tcol >= lo) & (tcol < hi), tcol - lo, 0)

    logits = jax.lax.dot_general(
        ph[...], w_ref[...], (((1,), (1,)), ((), ())),
        preferred_element_type=jnp.float32)
    logits = logits + b_ref[0, :, :]
    col_ids = j * cb + jax.lax.broadcasted_iota(jnp.int32, logits.shape, 1)
    logits = jnp.where(col_ids < n_valid, logits, -1e30)

    t[...] += jnp.sum(jnp.where(col_ids == idx, logits, 0.0),
                      axis=1, keepdims=True)
    bm = jnp.max(logits, axis=1, keepdims=True)
    m_new = jnp.maximum(m[...], bm)
    s[...] = (s[...] * jnp.exp(m[...] - m_new)
              + jnp.sum(jnp.exp(logits - m_new), axis=1, keepdims=True))
    m[...] = m_new

    @pl.when(j == ncb - 1)
    def _finish():
        nll = (m[...] + jnp.log(s[...])) - t[...]
        if lo is not None:
            inside = (tcol >= lo) & (tcol < hi)
            nll = jnp.where(inside, nll, 0.0)
        out_ref[...] = nll


def _cluster_nll(x, proj, w, b, tgtb, *, n_valid, cb, lo, hi, rb):
    n, d = x.shape
    p = proj.shape[1]
    nrb = n // rb
    ncb = pl.cdiv(n_valid, cb)
    bp = jnp.zeros((ncb * cb,), jnp.float32).at[:n_valid].set(b)
    bp = bp.reshape(ncb, 1, cb)

    body = functools.partial(_flash_nll_body, n_valid=n_valid, cb=cb,
                             ncb=ncb, lo=lo, hi=hi)
    out = pl.pallas_call(
        body,
        grid=(nrb, ncb),
        in_specs=[
            pl.BlockSpec((rb, d), lambda i, j: (i, 0)),
            pl.BlockSpec((d, p), lambda i, j: (0, 0)),
            pl.BlockSpec((cb, p), lambda i, j: (j, 0)),
            pl.BlockSpec((1, 1, cb), lambda i, j: (j, 0, 0)),
            pl.BlockSpec((rb, 128), lambda i, j: (i, 0)),
        ],
        out_specs=pl.BlockSpec((rb, 1), lambda i, j: (i, 0)),
        out_shape=jax.ShapeDtypeStruct((n, 1), jnp.float32),
        scratch_shapes=[
            pltpu.VMEM((rb, p), jnp.float32),
            pltpu.VMEM((rb, 1), jnp.float32),
            pltpu.VMEM((rb, 1), jnp.float32),
            pltpu.VMEM((rb, 1), jnp.float32),
        ],
    )(x, proj, w, bp, tgtb)
    return out[:, 0]


def kernel(input, target, cluster_weight, cluster_bias, proj0, proj1, proj2,
           w0, b0, w1, b1, w2, b2):
    n = input.shape[0]
    rb = 256
    tgtb = jnp.broadcast_to(target.astype(jnp.int32)[:, None], (n, 128))

    head_w = jnp.concatenate([w0, cluster_weight], axis=0)
    head_b = jnp.concatenate([b0, cluster_bias], axis=0)

    head = _cluster_nll(input, proj0, head_w, head_b, tgtb,
                        n_valid=head_w.shape[0], cb=1024,
                        lo=None, hi=None, rb=rb)
    t1 = _cluster_nll(input, proj1, w1, b1, tgtb,
                      n_valid=w1.shape[0], cb=2048,
                      lo=_CUT0, hi=_CUT1, rb=rb)
    t2 = _cluster_nll(input, proj2, w2, b2, tgtb,
                      n_valid=w2.shape[0], cb=2048,
                      lo=_CUT1, hi=_VOCAB, rb=rb)
    return head + t1 + t2


# speedup vs baseline: 1.6779x; 1.1573x over previous
import functools

import jax
import jax.numpy as jnp
from jax.experimental import pallas as pl
from jax.experimental.pallas import tpu as pltpu

_CUT0 = 20000
_CUT1 = 60000
_VOCAB = 100000


def _flash_nll_body(x_ref, proj_ref, w_ref, b_ref, tgt_ref, out_ref,
                    ph, m, s, t, *, n_valid, rb, cb, ncb, lo, hi):
    j = pl.program_id(0)
    i = pl.program_id(1)
    rows = pl.ds(i * rb, rb)

    @pl.when(j == 0)
    def _init():
        ph[rows, :] = jnp.dot(x_ref[...], proj_ref[...],
                              preferred_element_type=jnp.float32)
        m[rows, :] = jnp.full((rb, 1), -1e30, jnp.float32)
        s[rows, :] = jnp.zeros((rb, 1), jnp.float32)
        t[rows, :] = jnp.zeros((rb, 1), jnp.float32)

    tcol = tgt_ref[:, :1]
    if lo is None:
        idx = jnp.where(tcol >= _CUT1, _CUT0,
                        jnp.where(tcol >= _CUT0, _CUT0 + 1, tcol))
    else:
        idx = jnp.where((tcol >= lo) & (tcol < hi), tcol - lo, 0)

    logits = jax.lax.dot_general(
        ph[rows, :], w_ref[...], (((1,), (1,)), ((), ())),
        preferred_element_type=jnp.float32)
    logits = logits + b_ref[0, :, :]
    col_ids = j * cb + jax.lax.broadcasted_iota(jnp.int32, logits.shape, 1)
    logits = jnp.where(col_ids < n_valid, logits, -1e30)

    t[rows, :] += jnp.sum(jnp.where(col_ids == idx, logits, 0.0),
                          axis=1, keepdims=True)
    bm = jnp.max(logits, axis=1, keepdims=True)
    m_new = jnp.maximum(m[rows, :], bm)
    s[rows, :] = (s[rows, :] * jnp.exp(m[rows, :] - m_new)
                  + jnp.sum(jnp.exp(logits - m_new), axis=1, keepdims=True))
    m[rows, :] = m_new

    @pl.when(j == ncb - 1)
    def _finish():
        nll = (m[rows, :] + jnp.log(s[rows, :])) - t[rows, :]
        if lo is not None:
            inside = (tcol >= lo) & (tcol < hi)
            nll = jnp.where(inside, nll, 0.0)
        out_ref[rows, :] = nll


def _cluster_nll(x, proj, w, b, tgtb, *, n_valid, cb, lo, hi, rb):
    n, d = x.shape
    p = proj.shape[1]
    nrb = n // rb
    ncb = pl.cdiv(n_valid, cb)
    bp = jnp.zeros((ncb * cb,), jnp.float32).at[:n_valid].set(b)
    bp = bp.reshape(ncb, 1, cb)

    body = functools.partial(_flash_nll_body, n_valid=n_valid, rb=rb, cb=cb,
                             ncb=ncb, lo=lo, hi=hi)
    out = pl.pallas_call(
        body,
        grid=(ncb, nrb),
        in_specs=[
            pl.BlockSpec((rb, d), lambda j, i: (i, 0)),
            pl.BlockSpec((d, p), lambda j, i: (0, 0)),
            pl.BlockSpec((cb, p), lambda j, i: (j, 0)),
            pl.BlockSpec((1, 1, cb), lambda j, i: (j, 0, 0)),
            pl.BlockSpec((rb, 128), lambda j, i: (i, 0)),
        ],
        out_specs=pl.BlockSpec((n, 1), lambda j, i: (0, 0)),
        out_shape=jax.ShapeDtypeStruct((n, 1), jnp.float32),
        scratch_shapes=[
            pltpu.VMEM((n, p), jnp.float32),
            pltpu.VMEM((n, 1), jnp.float32),
            pltpu.VMEM((n, 1), jnp.float32),
            pltpu.VMEM((n, 1), jnp.float32),
        ],
        compiler_params=pltpu.CompilerParams(
            vmem_limit_bytes=100 * 1024 * 1024),
    )(x, proj, w, bp, tgtb)
    return out[:, 0]


def kernel(input, target, cluster_weight, cluster_bias, proj0, proj1, proj2,
           w0, b0, w1, b1, w2, b2):
    n = input.shape[0]
    rb = 256
    tgtb = jnp.broadcast_to(target.astype(jnp.int32)[:, None], (n, 128))

    head_w = jnp.concatenate([w0, cluster_weight], axis=0)
    head_b = jnp.concatenate([b0, cluster_bias], axis=0)

    head = _cluster_nll(input, proj0, head_w, head_b, tgtb,
                        n_valid=head_w.shape[0], cb=1024,
                        lo=None, hi=None, rb=rb)
    t1 = _cluster_nll(input, proj1, w1, b1, tgtb,
                      n_valid=w1.shape[0], cb=2048,
                      lo=_CUT0, hi=_CUT1, rb=rb)
    t2 = _cluster_nll(input, proj2, w2, b2, tgtb,
                      n_valid=w2.shape[0], cb=2048,
                      lo=_CUT1, hi=_VOCAB, rb=rb)
    return head + t1 + t2
